# baseline (speedup 1.0000x reference)
import functools

import numpy as np
import jax
import jax.numpy as jnp
from jax import lax
from jax.experimental import pallas as pl
from jax.experimental.pallas import tpu as pltpu
from jax.experimental.pallas import tpu_sc as plsc

_MIN_DISABLED = 4
_MAX_DISABLED = 16

_NUM_CORES = 2
_NUM_SUBCORES = 16
_LANES = 16
_CHUNK_ROWS = 16


def _disabled_tofs(tof_count: int) -> np.ndarray:
    rng = np.random.RandomState(0)
    disabled_count = int(rng.randint(_MIN_DISABLED, _MAX_DISABLED + 1))
    initial = int(rng.randint(0, tof_count))
    disabled = [initial]
    tof_list = rng.permutation(tof_count)
    tof_list = tof_list[tof_list != initial]
    for _ in range(disabled_count - 1):
        perm = rng.permutation(len(disabled))
        permuted = [disabled[i] for i in perm]
        opposite_found = False
        for cur in permuted:
            new_opp = (cur + tof_count // 2) % tof_count
            if new_opp not in disabled:
                disabled.append(int(new_opp))
                tof_list = tof_list[tof_list != new_opp]
                opposite_found = True
                break
        if not opposite_found:
            new_el = int(tof_list[0])
            tof_list = tof_list[tof_list != new_el]
            disabled.append(new_el)
    return np.asarray(disabled, dtype=np.int64)


def _make_sc_call(rows, cols):
    n_workers = _NUM_CORES * _NUM_SUBCORES
    rows_per_worker = rows // n_workers
    n_steps = rows_per_worker // _CHUNK_ROWS
    nbuf = 8
    assert rows_per_worker % _CHUNK_ROWS == 0 and n_steps % nbuf == 0

    mesh = plsc.VectorSubcoreMesh(core_axis_name="c", subcore_axis_name="s")

    @functools.partial(
        pl.kernel,
        out_type=jax.ShapeDtypeStruct((rows, cols), jnp.float32),
        mesh=mesh,
        scratch_types=[
            pltpu.VMEM((_LANES,), jnp.int32),
        ]
        + [pltpu.VMEM((_CHUNK_ROWS, cols), jnp.float32) for _ in range(nbuf)]
        + [pltpu.SemaphoreType.DMA for _ in range(2 * nbuf)],
        compiler_params=pltpu.CompilerParams(needs_layout_passes=False),
    )
    def sc_kernel(img_hbm, didx_hbm, out_hbm, idx_vm, *bufs_and_sems):
        bufs = bufs_and_sems[:nbuf]
        isems = bufs_and_sems[nbuf : 2 * nbuf]
        osems = bufs_and_sems[2 * nbuf : 3 * nbuf]
        wid = lax.axis_index("s") * _NUM_CORES + lax.axis_index("c")
        row0 = wid * rows_per_worker
        pltpu.sync_copy(didx_hbm, idx_vm)
        didx_const = idx_vm[...]
        zeros16 = jnp.zeros((_LANES,), dtype=jnp.float32)

        def copy_in(step, b):
            return pltpu.make_async_copy(
                img_hbm.at[pl.ds(row0 + step * _CHUNK_ROWS, _CHUNK_ROWS)],
                bufs[b],
                isems[b],
            )

        def copy_out(step, b):
            return pltpu.make_async_copy(
                bufs[b],
                out_hbm.at[pl.ds(row0 + step * _CHUNK_ROWS, _CHUNK_ROWS)],
                osems[b],
            )

        dist = 4
        for d in range(dist):
            copy_in(d, d).start()

        @pl.loop(0, n_steps, step=nbuf)
        def _body(step):
            for k in range(nbuf):
                i = step + k
                copy_in(i, k).wait()
                for r in range(_CHUNK_ROWS):
                    plsc.store_scatter(
                        bufs[k],
                        [jnp.full((_LANES,), r, jnp.int32), didx_const],
                        zeros16,
                    )
                copy_out(i, k).start()
                j = i + dist
                bj = (k + dist) % nbuf

                @pl.when(j < n_steps)
                def _refill():
                    @pl.when(j >= nbuf)
                    def _drain():
                        copy_out(j - nbuf, bj).wait()

                    copy_in(j, bj).start()

        for k in range(nbuf):
            copy_out(n_steps - nbuf + k, k).wait()

    return sc_kernel


def kernel(img) -> jnp.ndarray:
    rows, cols = img.shape
    disabled = _disabled_tofs(cols)
    didx = np.full((_LANES,), disabled[0], dtype=np.int32)
    didx[: len(disabled)] = disabled
    sc_call = _make_sc_call(rows, cols)
    return sc_call(img, jnp.asarray(didx))


# speedup vs baseline: 1.0047x; 1.0047x over previous
import functools

import numpy as np
import jax
import jax.numpy as jnp
from jax import lax
from jax.experimental import pallas as pl
from jax.experimental.pallas import tpu as pltpu
from jax.experimental.pallas import tpu_sc as plsc

_MIN_DISABLED = 4
_MAX_DISABLED = 16

_NUM_CORES = 2
_NUM_SUBCORES = 16
_LANES = 16
_CHUNK_ROWS = 32


def _disabled_tofs(tof_count: int) -> np.ndarray:
    rng = np.random.RandomState(0)
    disabled_count = int(rng.randint(_MIN_DISABLED, _MAX_DISABLED + 1))
    initial = int(rng.randint(0, tof_count))
    disabled = [initial]
    tof_list = rng.permutation(tof_count)
    tof_list = tof_list[tof_list != initial]
    for _ in range(disabled_count - 1):
        perm = rng.permutation(len(disabled))
        permuted = [disabled[i] for i in perm]
        opposite_found = False
        for cur in permuted:
            new_opp = (cur + tof_count // 2) % tof_count
            if new_opp not in disabled:
                disabled.append(int(new_opp))
                tof_list = tof_list[tof_list != new_opp]
                opposite_found = True
                break
        if not opposite_found:
            new_el = int(tof_list[0])
            tof_list = tof_list[tof_list != new_el]
            disabled.append(new_el)
    return np.asarray(disabled, dtype=np.int64)


def _disabled_windows(cols, disabled):
    windows = {}
    for c in sorted(int(c) for c in disabled):
        w0 = (c // _LANES) * _LANES
        windows.setdefault(w0, []).append(c - w0)
    return sorted((w0, tuple(ls)) for w0, ls in windows.items())


def _make_sc_call(rows, cols, windows):
    n_workers = _NUM_CORES * _NUM_SUBCORES
    rows_per_worker = rows // n_workers
    n_steps = rows_per_worker // _CHUNK_ROWS
    nbuf = 4
    dist = 2
    assert rows_per_worker % _CHUNK_ROWS == 0 and n_steps % nbuf == 0

    mesh = plsc.VectorSubcoreMesh(core_axis_name="c", subcore_axis_name="s")

    @functools.partial(
        pl.kernel,
        out_type=jax.ShapeDtypeStruct((rows, cols), jnp.float32),
        mesh=mesh,
        scratch_types=[
            pltpu.VMEM((_CHUNK_ROWS, cols), jnp.float32) for _ in range(nbuf)
        ]
        + [pltpu.SemaphoreType.DMA for _ in range(2 * nbuf)],
    )
    def sc_kernel(img_hbm, out_hbm, *bufs_and_sems):
        bufs = bufs_and_sems[:nbuf]
        isems = bufs_and_sems[nbuf : 2 * nbuf]
        osems = bufs_and_sems[2 * nbuf : 3 * nbuf]
        wid = lax.axis_index("s") * _NUM_CORES + lax.axis_index("c")
        row0 = wid * rows_per_worker

        def copy_in(step, b):
            return pltpu.make_async_copy(
                img_hbm.at[pl.ds(row0 + step * _CHUNK_ROWS, _CHUNK_ROWS)],
                bufs[b],
                isems[b],
            )

        def copy_out(step, b):
            return pltpu.make_async_copy(
                bufs[b],
                out_hbm.at[pl.ds(row0 + step * _CHUNK_ROWS, _CHUNK_ROWS)],
                osems[b],
            )

        lane_ids = lax.broadcasted_iota(jnp.int32, (_LANES,), 0)

        def zero_disabled(b):
            @pl.loop(0, _CHUNK_ROWS)
            def _rows(r):
                for w0, lanes in windows:
                    v = bufs[b][r, pl.ds(w0, _LANES)]
                    for l in lanes:
                        v = jnp.where(lane_ids == l, 0.0, v)
                    bufs[b][r, pl.ds(w0, _LANES)] = v

        for d in range(dist):
            copy_in(d, d).start()

        @pl.loop(0, n_steps, step=nbuf)
        def _body(step):
            for k in range(nbuf):
                i = step + k
                copy_in(i, k).wait()
                zero_disabled(k)
                copy_out(i, k).start()
                j = i + dist
                bj = (k + dist) % nbuf

                @pl.when(j < n_steps)
                def _refill():
                    @pl.when(j >= nbuf)
                    def _drain():
                        copy_out(j - nbuf, bj).wait()

                    copy_in(j, bj).start()

        for k in range(nbuf):
            copy_out(n_steps - nbuf + k, k).wait()

    return sc_kernel


def kernel(img) -> jnp.ndarray:
    rows, cols = img.shape
    disabled = _disabled_tofs(cols)
    windows = _disabled_windows(cols, disabled)
    sc_call = _make_sc_call(rows, cols, tuple(windows))
    return sc_call(img)
